# baseline (device time: 7046 ns/iter reference)
import jax
import jax.numpy as jnp
from jax import lax
from jax.experimental import pallas as pl
from jax.experimental.pallas import tpu as pltpu

N_CHUNK = 8


def kernel(x):
    m_per, n_per = x.shape
    rows = m_per // N_CHUNK

    def body(x_ref, out_ref, acc_ref, comm_ref, send_sem, recv_sem):
        i = pl.program_id(0)
        my_x = lax.axis_index("x")
        my_y = lax.axis_index("y")
        peer = (1 - my_x, my_y)

        @pl.when(i == 0)
        def _():
            barrier_sem = pltpu.get_barrier_semaphore()
            pl.semaphore_signal(
                barrier_sem, inc=1, device_id=peer,
                device_id_type=pl.DeviceIdType.MESH,
            )
            acc_ref[0, :] = jnp.sum(x_ref[:, :], axis=0)

        @pl.when(i > 0)
        def _():
            acc_ref[0, :] = acc_ref[0, :] + jnp.sum(x_ref[:, :], axis=0)

        @pl.when(i == N_CHUNK - 1)
        def _():
            barrier_sem = pltpu.get_barrier_semaphore()
            pl.semaphore_wait(barrier_sem, 1)

            rdma = pltpu.make_async_remote_copy(
                src_ref=acc_ref,
                dst_ref=comm_ref,
                send_sem=send_sem,
                recv_sem=recv_sem,
                device_id=peer,
                device_id_type=pl.DeviceIdType.MESH,
            )
            rdma.start()
            rdma.wait()
            out_ref[0, :] = acc_ref[0, :] + comm_ref[0, :]

    return pl.pallas_call(
        body,
        grid=(N_CHUNK,),
        out_shape=jax.ShapeDtypeStruct((1, n_per), jnp.float32),
        in_specs=[pl.BlockSpec((rows, n_per), lambda i: (i, 0))],
        out_specs=pl.BlockSpec((1, n_per), lambda i: (0, 0)),
        scratch_shapes=[
            pltpu.VMEM((1, n_per), jnp.float32),
            pltpu.VMEM((1, n_per), jnp.float32),
            pltpu.SemaphoreType.DMA,
            pltpu.SemaphoreType.DMA,
        ],
        compiler_params=pltpu.CompilerParams(collective_id=0),
    )(x)
